# baseline (speedup 1.0000x reference)
import functools

import jax
import jax.numpy as jnp
from jax import lax
from jax.experimental import pallas as pl
from jax.experimental.pallas import tpu as pltpu
from jax.experimental.pallas import tpu_sc as plsc

_NC = 2
_NS = 16
_NW = _NC * _NS
_P = 8


def _make_gather_sum(ntp, c, k):
    pk = _P * k
    n_per_w = ntp // _NW
    n_chunks = n_per_w // _P
    h = n_chunks // 2
    acc_rows = h * _P
    assert ntp % (_NW * _P) == 0 and c % 16 == 0 and n_chunks % 2 == 0
    zrows = acc_rows
    while zrows > pk or acc_rows % zrows:
        zrows //= 2

    mesh = plsc.VectorSubcoreMesh(
        core_axis_name="c", subcore_axis_name="s",
        num_cores=_NC, num_subcores=_NS)

    def body(table, idx, dst, out, idx_v, dst_v, buf_a, buf_b, out_b, acc,
             gs_a, gs_b, as_a, os_b):
        s = lax.axis_index("s")
        wid = s * _NC + lax.axis_index("c")
        row0 = wid * n_per_w
        sbase = s * acc_rows
        pltpu.sync_copy(idx.at[wid], idx_v)
        pltpu.sync_copy(dst.at[s], dst_v)

        z = jnp.zeros((16,), jnp.float32)
        for r in range(zrows):
            for cc in range(0, c, 16):
                buf_a[r, pl.ds(cc, 16)] = z
        for t in range(acc_rows // zrows):
            pltpu.sync_copy(buf_a.at[pl.ds(0, zrows)],
                            acc.at[pl.ds(sbase + t * zrows, zrows)])

        def fire_gather(ch, buf, sem):
            pltpu.async_copy(table.at[idx_v.at[ch]], buf, sem)

        def wait_gather(buf, sem):
            pltpu.make_async_copy(table.at[pl.ds(0, pk)], buf, sem).wait()

        def wait_out():
            pltpu.make_async_copy(out_b, out.at[pl.ds(row0, _P)], os_b).wait()

        def reduce(buf, obuf):
            for p in range(_P):
                base = p * k
                for cc in range(c // 16):
                    sl = pl.ds(cc * 16, 16)
                    v = [buf[base + j, sl] for j in range(k)]
                    while len(v) > 1:
                        nxt = [v[i] + v[i + 1] for i in range(0, len(v) - 1, 2)]
                        if len(v) % 2:
                            nxt.append(v[-1])
                        v = nxt
                    obuf[p, sl] = v[0]

        fire_gather(0, buf_a, gs_a)
        fire_gather(h, buf_b, gs_b)

        def step(i, carry):
            wait_gather(buf_a, gs_a)
            d_a = pltpu.async_copy(buf_a, acc.at[dst_v.at[i]], as_a,
                                   add=True)

            wait_gather(buf_b, gs_b)

            @pl.when(i > 0)
            def _():
                wait_out()

            reduce(buf_b, out_b)
            pltpu.async_copy(out_b, out.at[pl.ds(row0 + (h + i) * _P, _P)],
                             os_b)

            @pl.when(h + i + 1 < n_chunks)
            def _():
                fire_gather(h + i + 1, buf_b, gs_b)

            d_a.wait()

            @pl.when(i + 1 < h)
            def _():
                fire_gather(i + 1, buf_a, gs_a)

            return carry

        lax.fori_loop(0, h, step, 0)
        wait_out()
        pltpu.sync_copy(acc.at[pl.ds(sbase, acc_rows)],
                        out.at[pl.ds(row0, acc_rows)])

    return pl.kernel(
        body,
        out_type=jax.ShapeDtypeStruct((ntp, c), jnp.float32),
        mesh=mesh,
        scratch_types=[
            pltpu.VMEM((n_chunks, pk), jnp.int32),
            pltpu.VMEM((n_chunks // 2, pk), jnp.int32),
            pltpu.VMEM((pk, c), jnp.float32),
            pltpu.VMEM((pk, c), jnp.float32),
            pltpu.VMEM((_P, c), jnp.float32),
            pltpu.VMEM_SHARED((_NS * (n_chunks // 2) * _P, c), jnp.float32),
            pltpu.SemaphoreType.DMA,
            pltpu.SemaphoreType.DMA,
            pltpu.SemaphoreType.DMA,
            pltpu.SemaphoreType.DMA,
        ],
    )


def _relu_transpose(points_p, np_, c, nb):
    b = points_p.shape[0]

    def body(x_ref, o_ref):
        o_ref[0] = jnp.maximum(x_ref[0], 0.0).T

    return pl.pallas_call(
        body,
        grid=(b, np_ // nb),
        in_specs=[pl.BlockSpec((1, c, nb), lambda i, j: (i, 0, j))],
        out_specs=pl.BlockSpec((1, nb, c), lambda i, j: (i, j, 0)),
        out_shape=jax.ShapeDtypeStruct((b, np_, c), jnp.float32),
    )(points_p)


def _mix(points_p, g1, w1, w2, np_, c, nb, k):
    b = points_p.shape[0]
    inv = 1.0 / (k + 1.0)

    def body(x_ref, g_ref, w1_ref, w2_ref, o_ref):
        x = x_ref[0]
        y1 = lax.dot_general(jnp.maximum(x, 0.0), w1_ref[...],
                             (((0,), (1,)), ((), ())),
                             preferred_element_type=jnp.float32)
        y2 = lax.dot_general(g_ref[0], w2_ref[...],
                             (((1,), (1,)), ((), ())),
                             preferred_element_type=jnp.float32)
        o_ref[0] = jnp.maximum((y1 + y2) * inv + x.T, 0.0)

    return pl.pallas_call(
        body,
        grid=(b, np_ // nb),
        in_specs=[
            pl.BlockSpec((1, c, nb), lambda i, j: (i, 0, j)),
            pl.BlockSpec((1, nb, c), lambda i, j: (i, j, 0)),
            pl.BlockSpec((c, c), lambda i, j: (0, 0)),
            pl.BlockSpec((c, c), lambda i, j: (0, 0)),
        ],
        out_specs=pl.BlockSpec((1, nb, c), lambda i, j: (i, j, 0)),
        out_shape=jax.ShapeDtypeStruct((b, np_, c), jnp.float32),
    )(points_p, g1, w1, w2)


def _project(g2, wup_p, xyz12, np_, c, nb, k, oc):
    b = g2.shape[0]
    inv = 1.0 / k

    def body(g_ref, w_ref, x_ref, o_ref):
        y = lax.dot_general(g_ref[0], w_ref[...],
                            (((1,), (1,)), ((), ())),
                            preferred_element_type=jnp.float32)
        o_ref[0] = y * inv + x_ref[0]

    return pl.pallas_call(
        body,
        grid=(b, np_ // nb),
        in_specs=[
            pl.BlockSpec((1, nb, c), lambda i, j: (i, j, 0)),
            pl.BlockSpec((oc, c), lambda i, j: (0, 0)),
            pl.BlockSpec((1, nb, oc), lambda i, j: (i, j, 0)),
        ],
        out_specs=pl.BlockSpec((1, nb, oc), lambda i, j: (i, j, 0)),
        out_shape=jax.ShapeDtypeStruct((b, np_, oc), jnp.float32),
    )(g2, wup_p, xyz12)


def kernel(xyz, points, indices, W1, W2, Wup):
    b, c, n = points.shape
    k = indices.shape[2]
    oc = Wup.shape[0]
    up = oc // 3

    nb = 896
    np_ = ((n + nb - 1) // nb) * nb
    ntp = b * np_
    pad = np_ - n

    points_p = jnp.pad(points, ((0, 0), (0, 0), (0, pad)))
    idx_p = jnp.pad(indices, ((0, 0), (0, pad), (0, 0)))
    idx_g = idx_p + (jnp.arange(b, dtype=jnp.int32) * np_)[:, None, None]
    idx3d = idx_g.reshape(_NW, ntp // (_NW * _P), _P * k)

    n_per_w = ntp // _NW
    n_chunks = n_per_w // _P
    h = n_chunks // 2
    acc_rows = h * _P
    dst3d = (jnp.arange(_NS, dtype=jnp.int32)[:, None, None] * acc_rows
             + jnp.arange(h, dtype=jnp.int32)[None, :, None] * _P
             + jnp.repeat(jnp.arange(_P, dtype=jnp.int32), k)[None, None, :])

    gather_sum = _make_gather_sum(ntp, c, k)

    feats_t = _relu_transpose(points_p, np_, c, nb)
    g1 = gather_sum(feats_t.reshape(ntp, c), idx3d, dst3d).reshape(b, np_, c)
    f_t = _mix(points_p, g1, W1, W2, np_, c, nb, k)
    g2 = gather_sum(f_t.reshape(ntp, c), idx3d, dst3d).reshape(b, np_, c)

    wup_p = Wup.reshape(3, up, c).transpose(1, 0, 2).reshape(oc, c)
    xyz_p = jnp.pad(xyz, ((0, 0), (0, pad), (0, 0)))
    xyz12 = jnp.tile(xyz_p, (1, 1, up))
    out12 = _project(g2, wup_p, xyz12, np_, c, nb, k, oc)
    return out12[:, :n, :].reshape(b, n * up, 3)


# speedup vs baseline: 1.3651x; 1.3651x over previous
import functools

import jax
import jax.numpy as jnp
from jax import lax
from jax.experimental import pallas as pl
from jax.experimental.pallas import tpu as pltpu
from jax.experimental.pallas import tpu_sc as plsc

_NC = 2
_NS = 16
_NW = _NC * _NS
_P = 8


def _make_gather_sum(ntp, c, k):
    pk = _P * k
    n_per_w = ntp // _NW
    n_chunks = n_per_w // _P
    assert ntp % (_NW * _P) == 0 and c % 16 == 0

    mesh = plsc.VectorSubcoreMesh(
        core_axis_name="c", subcore_axis_name="s",
        num_cores=_NC, num_subcores=_NS)

    zrows = 112
    chz = zrows // _P
    s1 = (n_chunks // 2 // chz) * chz
    passes = [(0, s1), (s1, n_chunks - s1)]
    acc_rows = max(nch for _, nch in passes) * _P
    for _, nch in passes:
        assert nch % 2 == 0 and (nch * _P) % zrows == 0 and nch > 2

    def body(table, idx, dst, out, idx_v, dst_v, buf_a, buf_b, acc,
             gs_a, gs_b, as_a, as_b):
        s = lax.axis_index("s")
        wid = s * _NC + lax.axis_index("c")
        row0 = wid * n_per_w
        sbase = s * acc_rows
        pltpu.sync_copy(idx.at[wid], idx_v)
        pltpu.sync_copy(dst.at[s], dst_v)

        def fire_gather(ch, buf, sem):
            pltpu.async_copy(table.at[idx_v.at[ch]], buf, sem)

        def wait_gather(buf, sem):
            pltpu.make_async_copy(table.at[pl.ds(0, pk)], buf, sem).wait()

        for ch0, nch in passes:
            z = jnp.zeros((16,), jnp.float32)
            for r in range(zrows):
                for cc in range(0, c, 16):
                    buf_a[r, pl.ds(cc, 16)] = z
            for t in range((nch * _P) // zrows):
                pltpu.sync_copy(buf_a.at[pl.ds(0, zrows)],
                                acc.at[pl.ds(sbase + t * zrows, zrows)])

            fire_gather(ch0, buf_a, gs_a)
            fire_gather(ch0 + 1, buf_b, gs_b)

            def step(i, carry):
                c0 = ch0 + 2 * i
                wait_gather(buf_a, gs_a)
                pltpu.async_copy(buf_a, acc.at[dst_v.at[c0]], as_a,
                                 add=True).wait()

                @pl.when(c0 + 2 < ch0 + nch)
                def _():
                    fire_gather(c0 + 2, buf_a, gs_a)

                c1 = c0 + 1
                wait_gather(buf_b, gs_b)
                pltpu.async_copy(buf_b, acc.at[dst_v.at[c1]], as_b,
                                 add=True).wait()

                @pl.when(c1 + 2 < ch0 + nch)
                def _():
                    fire_gather(c1 + 2, buf_b, gs_b)

                return carry

            lax.fori_loop(0, nch // 2, step, 0)
            pltpu.sync_copy(acc.at[pl.ds(sbase, nch * _P)],
                            out.at[pl.ds(row0 + ch0 * _P, nch * _P)])

    return pl.kernel(
        body,
        out_type=jax.ShapeDtypeStruct((ntp, c), jnp.float32),
        mesh=mesh,
        scratch_types=[
            pltpu.VMEM((n_chunks, pk), jnp.int32),
            pltpu.VMEM((n_chunks, pk), jnp.int32),
            pltpu.VMEM((pk, c), jnp.float32),
            pltpu.VMEM((pk, c), jnp.float32),
            pltpu.VMEM_SHARED((_NS * acc_rows, c), jnp.float32),
            pltpu.SemaphoreType.DMA,
            pltpu.SemaphoreType.DMA,
            pltpu.SemaphoreType.DMA,
            pltpu.SemaphoreType.DMA,
        ],
    )


def _relu_transpose(points_p, np_, c, nb):
    b = points_p.shape[0]

    def body(x_ref, o_ref):
        o_ref[0] = jnp.maximum(x_ref[0], 0.0).T

    return pl.pallas_call(
        body,
        grid=(b, np_ // nb),
        in_specs=[pl.BlockSpec((1, c, nb), lambda i, j: (i, 0, j))],
        out_specs=pl.BlockSpec((1, nb, c), lambda i, j: (i, j, 0)),
        out_shape=jax.ShapeDtypeStruct((b, np_, c), jnp.float32),
    )(points_p)


def _mix(points_p, g1, w1, w2, np_, c, nb, k):
    b = points_p.shape[0]
    inv = 1.0 / (k + 1.0)

    def body(x_ref, g_ref, w1_ref, w2_ref, o_ref):
        x = x_ref[0]
        y1 = lax.dot_general(jnp.maximum(x, 0.0), w1_ref[...],
                             (((0,), (1,)), ((), ())),
                             preferred_element_type=jnp.float32)
        y2 = lax.dot_general(g_ref[0], w2_ref[...],
                             (((1,), (1,)), ((), ())),
                             preferred_element_type=jnp.float32)
        o_ref[0] = jnp.maximum((y1 + y2) * inv + x.T, 0.0)

    return pl.pallas_call(
        body,
        grid=(b, np_ // nb),
        in_specs=[
            pl.BlockSpec((1, c, nb), lambda i, j: (i, 0, j)),
            pl.BlockSpec((1, nb, c), lambda i, j: (i, j, 0)),
            pl.BlockSpec((c, c), lambda i, j: (0, 0)),
            pl.BlockSpec((c, c), lambda i, j: (0, 0)),
        ],
        out_specs=pl.BlockSpec((1, nb, c), lambda i, j: (i, j, 0)),
        out_shape=jax.ShapeDtypeStruct((b, np_, c), jnp.float32),
    )(points_p, g1, w1, w2)


def _project(g2, wup_p, xyz_p, np_, c, nb, k, oc):
    b = g2.shape[0]
    inv = 1.0 / k
    up = oc // 3

    def body(g_ref, w_ref, x_ref, o_ref):
        y = lax.dot_general(g_ref[0], w_ref[...],
                            (((1,), (1,)), ((), ())),
                            preferred_element_type=jnp.float32)
        x = x_ref[0]
        o_ref[0] = y * inv + jnp.concatenate([x] * up, axis=1)

    return pl.pallas_call(
        body,
        grid=(b, np_ // nb),
        in_specs=[
            pl.BlockSpec((1, nb, c), lambda i, j: (i, j, 0)),
            pl.BlockSpec((oc, c), lambda i, j: (0, 0)),
            pl.BlockSpec((1, nb, 3), lambda i, j: (i, j, 0)),
        ],
        out_specs=pl.BlockSpec((1, nb, oc), lambda i, j: (i, j, 0)),
        out_shape=jax.ShapeDtypeStruct((b, np_, oc), jnp.float32),
    )(g2, wup_p, xyz_p)


def kernel(xyz, points, indices, W1, W2, Wup):
    b, c, n = points.shape
    k = indices.shape[2]
    oc = Wup.shape[0]
    up = oc // 3

    nb = 896
    np_ = ((n + nb - 1) // nb) * nb
    ntp = b * np_
    pad = np_ - n

    points_p = jnp.pad(points, ((0, 0), (0, 0), (0, pad)))
    idx_p = jnp.pad(indices, ((0, 0), (0, pad), (0, 0)))
    idx_g = idx_p + (jnp.arange(b, dtype=jnp.int32) * np_)[:, None, None]
    idx3d = idx_g.reshape(_NW, ntp // (_NW * _P), _P * k)

    n_per_w = ntp // _NW
    n_chunks = n_per_w // _P
    chz = 112 // _P
    s1 = (n_chunks // 2 // chz) * chz
    acc_rows = max(s1, n_chunks - s1) * _P
    ch = jnp.arange(n_chunks, dtype=jnp.int32)
    lch = jnp.where(ch < s1, ch, ch - s1)
    dst3d = (jnp.arange(_NS, dtype=jnp.int32)[:, None, None] * acc_rows
             + lch[None, :, None] * _P
             + jnp.repeat(jnp.arange(_P, dtype=jnp.int32), k)[None, None, :])

    gather_sum = _make_gather_sum(ntp, c, k)

    feats_t = _relu_transpose(points_p, np_, c, nb)
    g1 = gather_sum(feats_t.reshape(ntp, c), idx3d, dst3d).reshape(b, np_, c)
    f_t = _mix(points_p, g1, W1, W2, np_, c, nb, k)
    g2 = gather_sum(f_t.reshape(ntp, c), idx3d, dst3d).reshape(b, np_, c)

    wup_p = Wup.reshape(3, up, c).transpose(1, 0, 2).reshape(oc, c)
    xyz_p = jnp.pad(xyz, ((0, 0), (0, pad), (0, 0)))
    out12 = _project(g2, wup_p, xyz_p, np_, c, nb, k, oc)
    return out12[:, :n, :].reshape(b, n * up, 3)


# speedup vs baseline: 1.4325x; 1.0494x over previous
import functools

import jax
import jax.numpy as jnp
from jax import lax
from jax.experimental import pallas as pl
from jax.experimental.pallas import tpu as pltpu
from jax.experimental.pallas import tpu_sc as plsc

_NC = 2
_NS = 16
_NW = _NC * _NS
_P = 8


def _make_gather_sum(ntp, c, k):
    pk = _P * k
    n_per_w = ntp // _NW
    n_chunks = n_per_w // _P
    assert ntp % (_NW * _P) == 0 and c % 16 == 0

    mesh = plsc.VectorSubcoreMesh(
        core_axis_name="c", subcore_axis_name="s",
        num_cores=_NC, num_subcores=_NS)

    zrows = 112
    chz = zrows // _P
    s1 = (n_chunks // 2 // chz) * chz
    passes = [(0, s1), (s1, n_chunks - s1)]
    acc_rows = max(nch for _, nch in passes) * _P
    for _, nch in passes:
        assert nch % 2 == 0 and (nch * _P) % zrows == 0 and nch > 2

    def body(table, idx, dst, out, idx_v, dst_v, buf_a, buf_b, acc,
             gs_a, gs_b, as_a, as_b):
        s = lax.axis_index("s")
        wid = s * _NC + lax.axis_index("c")
        row0 = wid * n_per_w
        sbase = s * acc_rows
        pltpu.sync_copy(idx.at[wid], idx_v)
        pltpu.sync_copy(dst.at[s], dst_v)

        def fire_gather(ch, buf, sem):
            pltpu.async_copy(table.at[idx_v.at[ch]], buf, sem)

        def wait_gather(buf, sem):
            pltpu.make_async_copy(table.at[pl.ds(0, pk)], buf, sem).wait()

        for ch0, nch in passes:
            z = jnp.zeros((16,), jnp.float32)
            for r in range(zrows):
                for cc in range(0, c, 16):
                    buf_a[r, pl.ds(cc, 16)] = z
            for t in range((nch * _P) // zrows):
                pltpu.sync_copy(buf_a.at[pl.ds(0, zrows)],
                                acc.at[pl.ds(sbase + t * zrows, zrows)])

            fire_gather(ch0, buf_a, gs_a)
            fire_gather(ch0 + 1, buf_b, gs_b)

            def step(i, carry):
                c0 = ch0 + 2 * i
                wait_gather(buf_a, gs_a)
                pltpu.async_copy(buf_a, acc.at[dst_v.at[c0]], as_a,
                                 add=True).wait()

                @pl.when(c0 + 2 < ch0 + nch)
                def _():
                    fire_gather(c0 + 2, buf_a, gs_a)

                c1 = c0 + 1
                wait_gather(buf_b, gs_b)
                pltpu.async_copy(buf_b, acc.at[dst_v.at[c1]], as_b,
                                 add=True).wait()

                @pl.when(c1 + 2 < ch0 + nch)
                def _():
                    fire_gather(c1 + 2, buf_b, gs_b)

                return carry

            lax.fori_loop(0, nch // 2, step, 0)
            pltpu.sync_copy(acc.at[pl.ds(sbase, nch * _P)],
                            out.at[pl.ds(row0 + ch0 * _P, nch * _P)])

    return pl.kernel(
        body,
        out_type=jax.ShapeDtypeStruct((ntp, c), jnp.float32),
        mesh=mesh,
        scratch_types=[
            pltpu.VMEM((n_chunks, pk), jnp.int32),
            pltpu.VMEM((n_chunks, pk), jnp.int32),
            pltpu.VMEM((pk, c), jnp.float32),
            pltpu.VMEM((pk, c), jnp.float32),
            pltpu.VMEM_SHARED((_NS * acc_rows, c), jnp.float32),
            pltpu.SemaphoreType.DMA,
            pltpu.SemaphoreType.DMA,
            pltpu.SemaphoreType.DMA,
            pltpu.SemaphoreType.DMA,
        ],
    )


def _relu_transpose(points_p, np_, c, nb):
    b = points_p.shape[0]

    def body(x_ref, o_ref):
        o_ref[0] = jnp.maximum(x_ref[0], 0.0).T

    return pl.pallas_call(
        body,
        grid=(b, np_ // nb),
        in_specs=[pl.BlockSpec((1, c, nb), lambda i, j: (i, 0, j))],
        out_specs=pl.BlockSpec((1, nb, c), lambda i, j: (i, j, 0)),
        out_shape=jax.ShapeDtypeStruct((b, np_, c), jnp.float32),
    )(points_p)


def _mix(points_p, g1, w1, w2, np_, c, nb, k):
    b = points_p.shape[0]
    inv = 1.0 / (k + 1.0)

    def body(x_ref, g_ref, w1_ref, w2_ref, o_ref):
        x = x_ref[0]
        y1 = lax.dot_general(jnp.maximum(x, 0.0), w1_ref[...],
                             (((0,), (1,)), ((), ())),
                             preferred_element_type=jnp.float32)
        y2 = lax.dot_general(g_ref[0], w2_ref[...],
                             (((1,), (1,)), ((), ())),
                             preferred_element_type=jnp.float32)
        o_ref[0] = jnp.maximum((y1 + y2) * inv + x.T, 0.0)

    return pl.pallas_call(
        body,
        grid=(b, np_ // nb),
        in_specs=[
            pl.BlockSpec((1, c, nb), lambda i, j: (i, 0, j)),
            pl.BlockSpec((1, nb, c), lambda i, j: (i, j, 0)),
            pl.BlockSpec((c, c), lambda i, j: (0, 0)),
            pl.BlockSpec((c, c), lambda i, j: (0, 0)),
        ],
        out_specs=pl.BlockSpec((1, nb, c), lambda i, j: (i, j, 0)),
        out_shape=jax.ShapeDtypeStruct((b, np_, c), jnp.float32),
    )(points_p, g1, w1, w2)


def _project(g2, wup_p, xyz12, np_, c, nb, k, oc):
    b = g2.shape[0]
    inv = 1.0 / k

    def body(g_ref, w_ref, x_ref, o_ref):
        y = lax.dot_general(g_ref[0], w_ref[...],
                            (((1,), (1,)), ((), ())),
                            preferred_element_type=jnp.float32)
        o_ref[0] = y * inv + x_ref[0]

    return pl.pallas_call(
        body,
        grid=(b, np_ // nb),
        in_specs=[
            pl.BlockSpec((1, nb, c), lambda i, j: (i, j, 0)),
            pl.BlockSpec((oc, c), lambda i, j: (0, 0)),
            pl.BlockSpec((1, nb, oc), lambda i, j: (i, j, 0)),
        ],
        out_specs=pl.BlockSpec((1, nb, oc), lambda i, j: (i, j, 0)),
        out_shape=jax.ShapeDtypeStruct((b, np_, oc), jnp.float32),
    )(g2, wup_p, xyz12)


def kernel(xyz, points, indices, W1, W2, Wup):
    b, c, n = points.shape
    k = indices.shape[2]
    oc = Wup.shape[0]
    up = oc // 3

    nb = 1792
    np_ = ((n + nb - 1) // nb) * nb
    ntp = b * np_
    pad = np_ - n

    points_p = jnp.pad(points, ((0, 0), (0, 0), (0, pad)))
    idx_p = jnp.pad(indices, ((0, 0), (0, pad), (0, 0)))
    idx_g = idx_p + (jnp.arange(b, dtype=jnp.int32) * np_)[:, None, None]
    idx3d = idx_g.reshape(_NW, ntp // (_NW * _P), _P * k)

    n_per_w = ntp // _NW
    n_chunks = n_per_w // _P
    chz = 112 // _P
    s1 = (n_chunks // 2 // chz) * chz
    acc_rows = max(s1, n_chunks - s1) * _P
    ch = jnp.arange(n_chunks, dtype=jnp.int32)
    lch = jnp.where(ch < s1, ch, ch - s1)
    dst3d = (jnp.arange(_NS, dtype=jnp.int32)[:, None, None] * acc_rows
             + lch[None, :, None] * _P
             + jnp.repeat(jnp.arange(_P, dtype=jnp.int32), k)[None, None, :])

    gather_sum = _make_gather_sum(ntp, c, k)

    feats_t = _relu_transpose(points_p, np_, c, nb)
    g1 = gather_sum(feats_t.reshape(ntp, c), idx3d, dst3d).reshape(b, np_, c)
    f_t = _mix(points_p, g1, W1, W2, np_, c, nb, k)
    g2 = gather_sum(f_t.reshape(ntp, c), idx3d, dst3d).reshape(b, np_, c)

    wup_p = Wup.reshape(3, up, c).transpose(1, 0, 2).reshape(oc, c)
    xyz_p = jnp.pad(xyz, ((0, 0), (0, pad), (0, 0)))
    xyz12 = jnp.tile(xyz_p, (1, 1, up))
    out12 = _project(g2, wup_p, xyz12, np_, c, nb, k, oc)
    return out12[:, :n, :].reshape(b, n * up, 3)


# speedup vs baseline: 1.4766x; 1.0308x over previous
import functools

import jax
import jax.numpy as jnp
from jax import lax
from jax.experimental import pallas as pl
from jax.experimental.pallas import tpu as pltpu
from jax.experimental.pallas import tpu_sc as plsc

_NC = 2
_NS = 16
_NW = _NC * _NS
_P = 8


def _make_gather_sum(ntp, c, k):
    pk = _P * k
    n_per_w = ntp // _NW
    n_chunks = n_per_w // _P
    assert ntp % (_NW * _P) == 0 and c % 16 == 0

    mesh = plsc.VectorSubcoreMesh(
        core_axis_name="c", subcore_axis_name="s",
        num_cores=_NC, num_subcores=_NS)

    zrows = 112
    chz = zrows // _P
    s1 = (n_chunks // 2 // chz) * chz
    passes = [(0, s1), (s1, n_chunks - s1)]
    acc_rows = max(nch for _, nch in passes) * _P
    for _, nch in passes:
        assert nch % 2 == 0 and (nch * _P) % zrows == 0 and nch > 2

    def body(table, idx, dst, out, idx_v, dst_v, buf_a, buf_b, acc,
             gs_a, gs_b, as_a, as_b):
        s = lax.axis_index("s")
        wid = s * _NC + lax.axis_index("c")
        row0 = wid * n_per_w
        sbase = s * acc_rows
        pltpu.sync_copy(idx.at[wid], idx_v)
        pltpu.sync_copy(dst.at[s], dst_v)

        def fire_gather(ch, buf, sem):
            pltpu.async_copy(table.at[idx_v.at[ch]], buf, sem)

        def wait_gather(buf, sem):
            pltpu.make_async_copy(table.at[pl.ds(0, pk)], buf, sem).wait()

        for ch0, nch in passes:
            z = jnp.zeros((16,), jnp.float32)
            for r in range(zrows):
                for cc in range(0, c, 16):
                    buf_a[r, pl.ds(cc, 16)] = z
            for t in range((nch * _P) // zrows):
                pltpu.sync_copy(buf_a.at[pl.ds(0, zrows)],
                                acc.at[pl.ds(sbase + t * zrows, zrows)])

            fire_gather(ch0, buf_a, gs_a)
            fire_gather(ch0 + 1, buf_b, gs_b)

            def step(i, carry):
                c0 = ch0 + 2 * i
                wait_gather(buf_a, gs_a)
                pltpu.async_copy(buf_a, acc.at[dst_v.at[c0]], as_a,
                                 add=True).wait()

                @pl.when(c0 + 2 < ch0 + nch)
                def _():
                    fire_gather(c0 + 2, buf_a, gs_a)

                c1 = c0 + 1
                wait_gather(buf_b, gs_b)
                pltpu.async_copy(buf_b, acc.at[dst_v.at[c1]], as_b,
                                 add=True).wait()

                @pl.when(c1 + 2 < ch0 + nch)
                def _():
                    fire_gather(c1 + 2, buf_b, gs_b)

                return carry

            lax.fori_loop(0, nch // 2, step, 0)
            pltpu.sync_copy(acc.at[pl.ds(sbase, nch * _P)],
                            out.at[pl.ds(row0 + ch0 * _P, nch * _P)])

    return pl.kernel(
        body,
        out_type=jax.ShapeDtypeStruct((ntp, c), jnp.float32),
        mesh=mesh,
        scratch_types=[
            pltpu.VMEM((n_chunks, pk), jnp.int32),
            pltpu.VMEM((n_chunks, pk), jnp.int32),
            pltpu.VMEM((pk, c), jnp.float32),
            pltpu.VMEM((pk, c), jnp.float32),
            pltpu.VMEM_SHARED((_NS * acc_rows, c), jnp.float32),
            pltpu.SemaphoreType.DMA,
            pltpu.SemaphoreType.DMA,
            pltpu.SemaphoreType.DMA,
            pltpu.SemaphoreType.DMA,
        ],
    )


def _relu_transpose(points_p, np_, c, nb):
    b = points_p.shape[0]

    def body(x_ref, o_ref):
        o_ref[0] = jnp.maximum(x_ref[0], 0.0).T

    return pl.pallas_call(
        body,
        grid=(b, np_ // nb),
        in_specs=[pl.BlockSpec((1, c, nb), lambda i, j: (i, 0, j))],
        out_specs=pl.BlockSpec((1, nb, c), lambda i, j: (i, j, 0)),
        out_shape=jax.ShapeDtypeStruct((b, np_, c), jnp.float32),
    )(points_p)


def _mix(points_p, g1, w1, w2, np_, c, nb, k):
    b = points_p.shape[0]
    inv = 1.0 / (k + 1.0)

    def body(x_ref, g_ref, w1_ref, w2_ref, o_ref):
        x = x_ref[0]
        y1 = lax.dot_general(jnp.maximum(x, 0.0), w1_ref[...],
                             (((0,), (1,)), ((), ())),
                             preferred_element_type=jnp.float32)
        y2 = lax.dot_general(g_ref[0], w2_ref[...],
                             (((1,), (1,)), ((), ())),
                             preferred_element_type=jnp.float32)
        o_ref[0] = jnp.maximum((y1 + y2) * inv + x.T, 0.0)

    return pl.pallas_call(
        body,
        grid=(b, np_ // nb),
        in_specs=[
            pl.BlockSpec((1, c, nb), lambda i, j: (i, 0, j)),
            pl.BlockSpec((1, nb, c), lambda i, j: (i, j, 0)),
            pl.BlockSpec((c, c), lambda i, j: (0, 0)),
            pl.BlockSpec((c, c), lambda i, j: (0, 0)),
        ],
        out_specs=pl.BlockSpec((1, nb, c), lambda i, j: (i, j, 0)),
        out_shape=jax.ShapeDtypeStruct((b, np_, c), jnp.float32),
    )(points_p, g1, w1, w2)


def _project(g2, wup_p, xyz12, np_, c, nb, k, oc):
    b = g2.shape[0]
    inv = 1.0 / k

    def body(g_ref, w_ref, x_ref, o_ref):
        y = lax.dot_general(g_ref[0], w_ref[...],
                            (((1,), (1,)), ((), ())),
                            preferred_element_type=jnp.float32)
        o_ref[0] = y * inv + x_ref[0]

    return pl.pallas_call(
        body,
        grid=(b, np_ // nb),
        in_specs=[
            pl.BlockSpec((1, nb, c), lambda i, j: (i, j, 0)),
            pl.BlockSpec((oc, c), lambda i, j: (0, 0)),
            pl.BlockSpec((1, nb, oc), lambda i, j: (i, j, 0)),
        ],
        out_specs=pl.BlockSpec((1, nb, oc), lambda i, j: (i, j, 0)),
        out_shape=jax.ShapeDtypeStruct((b, np_, oc), jnp.float32),
    )(g2, wup_p, xyz12)


def kernel(xyz, points, indices, W1, W2, Wup):
    b, c, n = points.shape
    k = indices.shape[2]
    oc = Wup.shape[0]
    up = oc // 3

    nb = 6272
    np_ = ((n + nb - 1) // nb) * nb
    ntp = b * np_
    pad = np_ - n

    points_p = jnp.pad(points, ((0, 0), (0, 0), (0, pad)))
    idx_p = jnp.pad(indices, ((0, 0), (0, pad), (0, 0)))
    idx_g = idx_p + (jnp.arange(b, dtype=jnp.int32) * np_)[:, None, None]
    idx3d = idx_g.reshape(_NW, ntp // (_NW * _P), _P * k)

    n_per_w = ntp // _NW
    n_chunks = n_per_w // _P
    chz = 112 // _P
    s1 = (n_chunks // 2 // chz) * chz
    acc_rows = max(s1, n_chunks - s1) * _P
    ch = jnp.arange(n_chunks, dtype=jnp.int32)
    lch = jnp.where(ch < s1, ch, ch - s1)
    dst3d = (jnp.arange(_NS, dtype=jnp.int32)[:, None, None] * acc_rows
             + lch[None, :, None] * _P
             + jnp.repeat(jnp.arange(_P, dtype=jnp.int32), k)[None, None, :])

    gather_sum = _make_gather_sum(ntp, c, k)

    feats_t = _relu_transpose(points_p, np_, c, nb)
    g1 = gather_sum(feats_t.reshape(ntp, c), idx3d, dst3d).reshape(b, np_, c)
    f_t = _mix(points_p, g1, W1, W2, np_, c, nb, k)
    g2 = gather_sum(f_t.reshape(ntp, c), idx3d, dst3d).reshape(b, np_, c)

    wup_p = Wup.reshape(3, up, c).transpose(1, 0, 2).reshape(oc, c)
    xyz_p = jnp.pad(xyz, ((0, 0), (0, pad), (0, 0)))
    xyz12 = jnp.tile(xyz_p, (1, 1, up))
    out12 = _project(g2, wup_p, xyz12, np_, c, nb, k, oc)
    return out12[:, :n, :].reshape(b, n * up, 3)


# speedup vs baseline: 1.4832x; 1.0045x over previous
import functools

import jax
import jax.numpy as jnp
from jax import lax
from jax.experimental import pallas as pl
from jax.experimental.pallas import tpu as pltpu
from jax.experimental.pallas import tpu_sc as plsc

_NC = 2
_NS = 16
_NW = _NC * _NS
_P = 8


def _make_gather_sum(ntp, c, k):
    pk = _P * k
    n_per_w = ntp // _NW
    n_chunks = n_per_w // _P
    assert ntp % (_NW * _P) == 0 and c % 16 == 0

    mesh = plsc.VectorSubcoreMesh(
        core_axis_name="c", subcore_axis_name="s",
        num_cores=_NC, num_subcores=_NS)

    zrows = 112
    chz = zrows // _P
    s1 = (n_chunks // 2 // chz) * chz
    passes = [(0, s1), (s1, n_chunks - s1)]
    acc_rows = max(nch for _, nch in passes) * _P
    for _, nch in passes:
        assert nch % 2 == 0 and (nch * _P) % zrows == 0 and nch > 2

    def body(table, idx, dst, out, idx_v, dst_v, buf_a, buf_b, acc,
             gs_a, gs_b, as_a, as_b):
        s = lax.axis_index("s")
        wid = s * _NC + lax.axis_index("c")
        row0 = wid * n_per_w
        sbase = s * acc_rows
        pltpu.sync_copy(idx.at[wid], idx_v)
        pltpu.sync_copy(dst.at[s], dst_v)

        def fire_gather(ch, buf, sem):
            pltpu.async_copy(table.at[idx_v.at[ch]], buf, sem)

        def wait_gather(buf, sem):
            pltpu.make_async_copy(table.at[pl.ds(0, pk)], buf, sem).wait()

        for ch0, nch in passes:
            z = jnp.zeros((16,), jnp.float32)
            for r in range(zrows):
                for cc in range(0, c, 16):
                    buf_a[r, pl.ds(cc, 16)] = z
            for t in range((nch * _P) // zrows):
                pltpu.sync_copy(buf_a.at[pl.ds(0, zrows)],
                                acc.at[pl.ds(sbase + t * zrows, zrows)])

            fire_gather(ch0, buf_a, gs_a)
            fire_gather(ch0 + 1, buf_b, gs_b)

            def step(i, carry):
                c0 = ch0 + 2 * i
                wait_gather(buf_a, gs_a)
                pltpu.async_copy(buf_a, acc.at[dst_v.at[c0]], as_a,
                                 add=True).wait()

                @pl.when(c0 + 2 < ch0 + nch)
                def _():
                    fire_gather(c0 + 2, buf_a, gs_a)

                c1 = c0 + 1
                wait_gather(buf_b, gs_b)
                pltpu.async_copy(buf_b, acc.at[dst_v.at[c1]], as_b,
                                 add=True).wait()

                @pl.when(c1 + 2 < ch0 + nch)
                def _():
                    fire_gather(c1 + 2, buf_b, gs_b)

                return carry

            lax.fori_loop(0, nch // 2, step, 0)
            pltpu.sync_copy(acc.at[pl.ds(sbase, nch * _P)],
                            out.at[pl.ds(row0 + ch0 * _P, nch * _P)])

    return pl.kernel(
        body,
        out_type=jax.ShapeDtypeStruct((ntp, c), jnp.float32),
        mesh=mesh,
        scratch_types=[
            pltpu.VMEM((n_chunks, pk), jnp.int32),
            pltpu.VMEM((n_chunks, pk), jnp.int32),
            pltpu.VMEM((pk, c), jnp.float32),
            pltpu.VMEM((pk, c), jnp.float32),
            pltpu.VMEM_SHARED((_NS * acc_rows, c), jnp.float32),
            pltpu.SemaphoreType.DMA,
            pltpu.SemaphoreType.DMA,
            pltpu.SemaphoreType.DMA,
            pltpu.SemaphoreType.DMA,
        ],
    )


def _relu_transpose(points_p, np_, c, nb):
    b = points_p.shape[0]

    def body(x_ref, o_ref):
        o_ref[0] = jnp.maximum(x_ref[0], 0.0).T

    return pl.pallas_call(
        body,
        grid=(b, np_ // nb),
        in_specs=[pl.BlockSpec((1, c, nb), lambda i, j: (i, 0, j))],
        out_specs=pl.BlockSpec((1, nb, c), lambda i, j: (i, j, 0)),
        out_shape=jax.ShapeDtypeStruct((b, np_, c), jnp.float32),
    )(points_p)


def _mix(points_p, g1, w1, w2, np_, c, nb, k):
    b = points_p.shape[0]
    inv = 1.0 / (k + 1.0)

    def body(x_ref, g_ref, w1_ref, w2_ref, o_ref):
        x = x_ref[0]
        y1 = lax.dot_general(jnp.maximum(x, 0.0), w1_ref[...],
                             (((0,), (1,)), ((), ())),
                             preferred_element_type=jnp.float32)
        y2 = lax.dot_general(g_ref[0], w2_ref[...],
                             (((1,), (1,)), ((), ())),
                             preferred_element_type=jnp.float32)
        o_ref[0] = jnp.maximum((y1 + y2) * inv + x.T, 0.0)

    return pl.pallas_call(
        body,
        grid=(b, np_ // nb),
        in_specs=[
            pl.BlockSpec((1, c, nb), lambda i, j: (i, 0, j)),
            pl.BlockSpec((1, nb, c), lambda i, j: (i, j, 0)),
            pl.BlockSpec((c, c), lambda i, j: (0, 0)),
            pl.BlockSpec((c, c), lambda i, j: (0, 0)),
        ],
        out_specs=pl.BlockSpec((1, nb, c), lambda i, j: (i, j, 0)),
        out_shape=jax.ShapeDtypeStruct((b, np_, c), jnp.float32),
    )(points_p, g1, w1, w2)


def _project(g2, wup_p, xyz12, np_, c, nb, k, oc):
    b = g2.shape[0]
    inv = 1.0 / k

    def body(g_ref, w_ref, x_ref, o_ref):
        y = lax.dot_general(g_ref[0], w_ref[...],
                            (((1,), (1,)), ((), ())),
                            preferred_element_type=jnp.float32)
        o_ref[0] = y * inv + x_ref[0]

    return pl.pallas_call(
        body,
        grid=(b, np_ // nb),
        in_specs=[
            pl.BlockSpec((1, nb, c), lambda i, j: (i, j, 0)),
            pl.BlockSpec((oc, c), lambda i, j: (0, 0)),
            pl.BlockSpec((1, nb, oc), lambda i, j: (i, j, 0)),
        ],
        out_specs=pl.BlockSpec((1, nb, oc), lambda i, j: (i, j, 0)),
        out_shape=jax.ShapeDtypeStruct((b, np_, oc), jnp.float32),
    )(g2, wup_p, xyz12)


def kernel(xyz, points, indices, W1, W2, Wup):
    b, c, n = points.shape
    k = indices.shape[2]
    oc = Wup.shape[0]
    up = oc // 3

    nb = 12544
    np_ = ((n + nb - 1) // nb) * nb
    ntp = b * np_
    pad = np_ - n

    points_p = jnp.pad(points, ((0, 0), (0, 0), (0, pad)))
    idx_p = jnp.pad(indices, ((0, 0), (0, pad), (0, 0)))
    idx_g = idx_p + (jnp.arange(b, dtype=jnp.int32) * np_)[:, None, None]
    idx3d = idx_g.reshape(_NW, ntp // (_NW * _P), _P * k)

    n_per_w = ntp // _NW
    n_chunks = n_per_w // _P
    chz = 112 // _P
    s1 = (n_chunks // 2 // chz) * chz
    acc_rows = max(s1, n_chunks - s1) * _P
    ch = jnp.arange(n_chunks, dtype=jnp.int32)
    lch = jnp.where(ch < s1, ch, ch - s1)
    dst3d = (jnp.arange(_NS, dtype=jnp.int32)[:, None, None] * acc_rows
             + lch[None, :, None] * _P
             + jnp.repeat(jnp.arange(_P, dtype=jnp.int32), k)[None, None, :])

    gather_sum = _make_gather_sum(ntp, c, k)

    feats_t = _relu_transpose(points_p, np_, c, nb)
    g1 = gather_sum(feats_t.reshape(ntp, c), idx3d, dst3d).reshape(b, np_, c)
    f_t = _mix(points_p, g1, W1, W2, np_, c, nb, k)
    g2 = gather_sum(f_t.reshape(ntp, c), idx3d, dst3d).reshape(b, np_, c)

    wup_p = Wup.reshape(3, up, c).transpose(1, 0, 2).reshape(oc, c)
    xyz_p = jnp.pad(xyz, ((0, 0), (0, pad), (0, 0)))
    xyz12 = jnp.tile(xyz_p, (1, 1, up))
    out12 = _project(g2, wup_p, xyz12, np_, c, nb, k, oc)
    return out12[:, :n, :].reshape(b, n * up, 3)
